# speedup vs baseline: 18.6452x; 18.6452x over previous
import jax
import jax.numpy as jnp
from jax import lax
from jax.experimental import pallas as pl
from jax.experimental.pallas import tpu as pltpu

_N = 32768
_K = 512
_BOOST = 1.0
_ROWS = 128
_RB = 16


def _mask_body(x_ref, duty_ref, out_ref):
    x = x_ref[...]
    duty = duty_ref[...]
    factors = jnp.exp((jnp.float32(_K) / jnp.float32(_N) - duty) * jnp.float32(_BOOST))
    boosted = x * factors
    i = lax.bitcast_convert_type(boosted, jnp.int32)
    key_s = i ^ ((i >> 31) & jnp.int32(0x7FFFFFFF))
    ku = lax.bitcast_convert_type(key_s, jnp.uint32) ^ jnp.uint32(0x80000000)

    def body(j, t):
        b = jnp.uint32(31) - j.astype(jnp.uint32)
        cand = t | jnp.left_shift(jnp.uint32(1), b)
        cnt = jnp.sum((ku >= cand).astype(jnp.int32), axis=1, keepdims=True)
        return jnp.where(cnt >= _K, cand, t)

    t0 = jnp.zeros((x.shape[0], 1), jnp.uint32)
    T = lax.fori_loop(0, 32, body, t0)
    out_ref[...] = (ku >= T).astype(jnp.float32)


def kernel(x, dutyCycle):
    duty2d = dutyCycle.reshape(1, _N)
    return pl.pallas_call(
        _mask_body,
        grid=(_ROWS // _RB,),
        in_specs=[
            pl.BlockSpec((_RB, _N), lambda r: (r, 0)),
            pl.BlockSpec((1, _N), lambda r: (0, 0)),
        ],
        out_specs=pl.BlockSpec((_RB, _N), lambda r: (r, 0)),
        out_shape=jax.ShapeDtypeStruct((_ROWS, _N), jnp.float32),
    )(x, duty2d)
